# Initial kernel scaffold; baseline (speedup 1.0000x reference)
import jax
import jax.numpy as jnp
from jax.experimental import pallas as pl


def kernel(x):
    raise NotImplementedError("write your pallas kernel here")


# speedup vs baseline: 12.4885x; 12.4885x over previous
import math

import jax
import jax.numpy as jnp
from jax.experimental import pallas as pl
from jax.experimental.pallas import tpu as pltpu

_SPARSITY = 0.05
_ROW_BLOCK = 32


_SIGN = -0x80000000


def _kwta_body(x_ref, o_ref, *, k_active):
    x = x_ref[...]
    bi = jax.lax.bitcast_convert_type(x, jnp.int32)
    skeys = jnp.where(bi < 0, ~bi ^ jnp.int32(_SIGN), bi)

    rows = x.shape[0]
    kb = jnp.zeros((rows, 1), jnp.int32)
    for bit in range(31, -1, -1):
        cand_b = kb | jnp.int32(_SIGN if bit == 31 else (1 << bit))
        cand_s = cand_b ^ jnp.int32(_SIGN)
        cnt = jnp.sum((skeys >= cand_s).astype(jnp.int32), axis=1,
                      keepdims=True)
        kb = jnp.where(cnt >= k_active, cand_b, kb)
    kk = kb ^ jnp.int32(_SIGN)

    cnt_at_k = jnp.sum((skeys >= kk).astype(jnp.int32), axis=1, keepdims=True)
    below = jnp.where(skeys < kk, skeys, jnp.int32(_SIGN))
    max_below = jnp.max(below, axis=1, keepdims=True)
    kk2 = jnp.where(cnt_at_k >= k_active + 1, kk, max_below)

    def key_to_float(key):
        fb = jnp.where(key < 0, ~(key ^ jnp.int32(_SIGN)), key)
        return jax.lax.bitcast_convert_type(fb, jnp.float32)

    threshold = (key_to_float(kk) + key_to_float(kk2)) * jnp.float32(0.5)
    o_ref[...] = (x > threshold).astype(jnp.float32)


def kernel(x):
    batch, dim = x.shape
    k_active = math.ceil(_SPARSITY * dim)
    if k_active == dim:
        k_active -= 1
    import functools
    body = functools.partial(_kwta_body, k_active=k_active)
    grid = (batch // _ROW_BLOCK,)
    return pl.pallas_call(
        body,
        grid=grid,
        in_specs=[pl.BlockSpec((_ROW_BLOCK, dim), lambda i: (i, 0))],
        out_specs=pl.BlockSpec((_ROW_BLOCK, dim), lambda i: (i, 0)),
        out_shape=jax.ShapeDtypeStruct((batch, dim), jnp.float32),
        compiler_params=pltpu.CompilerParams(
            dimension_semantics=("arbitrary",),
        ),
    )(x)
